# Initial kernel scaffold; baseline (speedup 1.0000x reference)
import functools

import jax
import jax.numpy as jnp
from jax import lax
from jax.experimental import pallas as pl
from jax.experimental.pallas import tpu as pltpu
from jax.experimental.pallas import tpu_sc as plsc

N = 10000
E = 320000
D_IN = 128
HID = 64
NCLS = 40
F2 = 48

NC = 2
NS = 16
NW = NC * NS

CHUNK = 128
NCHUNK = 80
EPAD = NW * NCHUNK * CHUNK
NPAD = 10240
ROWS_PER_TILE = NPAD // NS

_MESH = plsc.VectorSubcoreMesh(core_axis_name="c", subcore_axis_name="s")


def _fill_rows(buf, f, val):
    def body(i, _):
        for k in range(f // 16):
            buf[i, pl.ds(k * 16, 16)] = jnp.full((16,), val, jnp.float32)
        return 0
    lax.fori_loop(0, CHUNK, body, 0)


def _deg_body(dst_hbm, out_hbm, dstv, onesv, accum, sem):
    c = lax.axis_index("c")
    s = lax.axis_index("s")
    wid = c * NS + s

    _fill_rows(onesv, 16, 0.0)

    def zloop(i, _):
        pltpu.sync_copy(onesv, accum.at[pl.ds(s * ROWS_PER_TILE + i * CHUNK, CHUNK)])
        return 0
    lax.fori_loop(0, ROWS_PER_TILE // CHUNK, zloop, 0)

    _fill_rows(onesv, 16, 1.0)
    pltpu.sync_copy(dst_hbm.at[wid], dstv)
    plsc.subcore_barrier()

    def body(j, _):
        pltpu.sync_copy(onesv, accum.at[dstv.at[j]], add=True)
        return 0
    lax.fori_loop(0, NCHUNK, body, 0)

    plsc.subcore_barrier()
    pltpu.sync_copy(
        accum.at[pl.ds(s * ROWS_PER_TILE, ROWS_PER_TILE)],
        out_hbm.at[c, pl.ds(s * ROWS_PER_TILE, ROWS_PER_TILE)],
    )


_deg_kernel = pl.kernel(
    _deg_body,
    out_type=jax.ShapeDtypeStruct((NC, NPAD, 16), jnp.float32),
    mesh=_MESH,
    scratch_types=[
        pltpu.VMEM((NCHUNK, CHUNK), jnp.int32),
        pltpu.VMEM((CHUNK, 16), jnp.float32),
        pltpu.VMEM_SHARED((NPAD, 16), jnp.float32),
        pltpu.SemaphoreType.DMA,
    ],
)


def _make_scatter_kernel(f):
    def body(table_hbm, src_hbm, dst_hbm, out_hbm,
             srcv, dstv, rows, zbuf, accum, sem):
        c = lax.axis_index("c")
        s = lax.axis_index("s")
        wid = c * NS + s

        _fill_rows(zbuf, f, 0.0)

        def zloop(i, _):
            pltpu.sync_copy(zbuf, accum.at[pl.ds(s * ROWS_PER_TILE + i * CHUNK, CHUNK)])
            return 0
        lax.fori_loop(0, ROWS_PER_TILE // CHUNK, zloop, 0)

        pltpu.sync_copy(src_hbm.at[wid], srcv)
        pltpu.sync_copy(dst_hbm.at[wid], dstv)
        plsc.subcore_barrier()

        def body_loop(j, _):
            pltpu.async_copy(table_hbm.at[srcv.at[j]], rows, sem).wait()
            pltpu.sync_copy(rows, accum.at[dstv.at[j]], add=True)
            return 0
        lax.fori_loop(0, NCHUNK, body_loop, 0)

        plsc.subcore_barrier()
        pltpu.sync_copy(
            accum.at[pl.ds(s * ROWS_PER_TILE, ROWS_PER_TILE)],
            out_hbm.at[c, pl.ds(s * ROWS_PER_TILE, ROWS_PER_TILE)],
        )

    return pl.kernel(
        body,
        out_type=jax.ShapeDtypeStruct((NC, NPAD, f), jnp.float32),
        mesh=_MESH,
        scratch_types=[
            pltpu.VMEM((NCHUNK, CHUNK), jnp.int32),
            pltpu.VMEM((NCHUNK, CHUNK), jnp.int32),
            pltpu.VMEM((CHUNK, f), jnp.float32),
            pltpu.VMEM((CHUNK, f), jnp.float32),
            pltpu.VMEM_SHARED((NPAD, f), jnp.float32),
            pltpu.SemaphoreType.DMA,
        ],
    )


_scatter_hid = _make_scatter_kernel(HID)
_scatter_f2 = _make_scatter_kernel(F2)


RBLK = 2000
GRID = N // RBLK


def _dinv_from(degp_ref):
    deg = degp_ref[0, :, 0:1] + degp_ref[1, :, 0:1] + 1.0
    return lax.rsqrt(deg)


def _tc1_body(x_ref, w1_ref, degp_ref, g1_ref):
    dinv = _dinv_from(degp_ref)
    h = jnp.dot(x_ref[...], w1_ref[...], preferred_element_type=jnp.float32)
    g1_ref[...] = h * dinv


_tc1 = pl.pallas_call(
    _tc1_body,
    grid=(GRID,),
    in_specs=[
        pl.BlockSpec((RBLK, D_IN), lambda i: (i, 0)),
        pl.BlockSpec((D_IN, HID), lambda i: (0, 0)),
        pl.BlockSpec((NC, RBLK, 16), lambda i: (0, i, 0)),
    ],
    out_specs=pl.BlockSpec((RBLK, HID), lambda i: (i, 0)),
    out_shape=jax.ShapeDtypeStruct((N, HID), jnp.float32),
)


def _tc2_body(p_ref, g1_ref, degp_ref, b1_ref, w2_ref, g2_ref):
    dinv = _dinv_from(degp_ref)
    agg = p_ref[0] + p_ref[1] + g1_ref[...]
    h1 = jnp.maximum(dinv * agg + b1_ref[...], 0.0)
    h2 = jnp.dot(h1, w2_ref[...], preferred_element_type=jnp.float32)
    g2_ref[...] = h2 * dinv


_tc2 = pl.pallas_call(
    _tc2_body,
    grid=(GRID,),
    in_specs=[
        pl.BlockSpec((NC, RBLK, HID), lambda i: (0, i, 0)),
        pl.BlockSpec((RBLK, HID), lambda i: (i, 0)),
        pl.BlockSpec((NC, RBLK, 16), lambda i: (0, i, 0)),
        pl.BlockSpec((1, HID), lambda i: (0, 0)),
        pl.BlockSpec((HID, F2), lambda i: (0, 0)),
    ],
    out_specs=pl.BlockSpec((RBLK, F2), lambda i: (i, 0)),
    out_shape=jax.ShapeDtypeStruct((N, F2), jnp.float32),
)


def _tc3_body(p_ref, g2_ref, degp_ref, b2_ref, out_ref):
    dinv = _dinv_from(degp_ref)
    z = dinv * (p_ref[0] + p_ref[1] + g2_ref[...]) + b2_ref[...]
    z = z[:, :NCLS]
    m = jnp.max(z, axis=1, keepdims=True)
    zs = z - m
    out_ref[...] = zs - jnp.log(jnp.sum(jnp.exp(zs), axis=1, keepdims=True))


_tc3 = pl.pallas_call(
    _tc3_body,
    grid=(GRID,),
    in_specs=[
        pl.BlockSpec((NC, RBLK, F2), lambda i: (0, i, 0)),
        pl.BlockSpec((RBLK, F2), lambda i: (i, 0)),
        pl.BlockSpec((NC, RBLK, 16), lambda i: (0, i, 0)),
        pl.BlockSpec((1, F2), lambda i: (0, 0)),
    ],
    out_specs=pl.BlockSpec((RBLK, NCLS), lambda i: (i, 0)),
    out_shape=jax.ShapeDtypeStruct((N, NCLS), jnp.float32),
)


@jax.jit
def kernel(x, edge_index, W1, b1, W2, b2):
    src = edge_index[0].astype(jnp.int32)
    dst = edge_index[1].astype(jnp.int32)
    src_p = jnp.concatenate([src, jnp.zeros((EPAD - E,), jnp.int32)])
    dst_p = jnp.concatenate([dst, jnp.full((EPAD - E,), NPAD - 1, jnp.int32)])
    src_r = src_p.reshape(NW, NCHUNK, CHUNK)
    dst_r = dst_p.reshape(NW, NCHUNK, CHUNK)

    w2p = jnp.pad(W2, ((0, 0), (0, F2 - NCLS)))
    b1r = b1.reshape(1, HID)
    b2r = jnp.pad(b2, (0, F2 - NCLS)).reshape(1, F2)

    degp = _deg_kernel(dst_r)
    g1 = _tc1(x, W1, degp)
    p1 = _scatter_hid(g1, src_r, dst_r)
    g2 = _tc2(p1, g1, degp, b1r, w2p)
    p2 = _scatter_f2(g2, src_r, dst_r)
    return _tc3(p2, g2, degp, b2r)


# speedup vs baseline: 15.5211x; 15.5211x over previous
import functools

import jax
import jax.numpy as jnp
from jax import lax
from jax.experimental import pallas as pl
from jax.experimental.pallas import tpu as pltpu
from jax.experimental.pallas import tpu_sc as plsc

N = 10000
E = 320000
D_IN = 128
HID = 64
NCLS = 40
F2 = 48

NC = 2
NS = 16
NW = NC * NS

CHUNK = 128
NCHUNK = 80
EPAD = NW * NCHUNK * CHUNK
NPAD = 10240
ROWS_PER_TILE = NPAD // NS

_MESH = plsc.VectorSubcoreMesh(
    core_axis_name="c", subcore_axis_name="s", num_cores=NC, num_subcores=NS
)


def _fill_rows(buf, f, val):
    def body(i, _):
        for k in range(f // 16):
            buf[i, pl.ds(k * 16, 16)] = jnp.full((16,), val, jnp.float32)
        return 0
    lax.fori_loop(0, CHUNK, body, 0)


def _deg_body(dst_hbm, out_hbm, dstv, onesv, accum, sem):
    c = lax.axis_index("c")
    s = lax.axis_index("s")
    wid = c * NS + s

    _fill_rows(onesv, 16, 0.0)

    def zloop(i, _):
        pltpu.sync_copy(onesv, accum.at[pl.ds(s * ROWS_PER_TILE + i * CHUNK, CHUNK)])
        return 0
    lax.fori_loop(0, ROWS_PER_TILE // CHUNK, zloop, 0)

    _fill_rows(onesv, 16, 1.0)
    pltpu.sync_copy(dst_hbm.at[wid], dstv)
    plsc.subcore_barrier()

    def body(j, _):
        pltpu.sync_copy(onesv, accum.at[dstv.at[j]], add=True)
        return 0
    lax.fori_loop(0, NCHUNK, body, 0)

    plsc.subcore_barrier()
    pltpu.sync_copy(
        accum.at[pl.ds(s * ROWS_PER_TILE, ROWS_PER_TILE)],
        out_hbm.at[c, pl.ds(s * ROWS_PER_TILE, ROWS_PER_TILE)],
    )


_deg_kernel = pl.kernel(
    _deg_body,
    out_type=jax.ShapeDtypeStruct((NC, NPAD, 16), jnp.float32),
    mesh=_MESH,
    compiler_params=pltpu.CompilerParams(use_tc_tiling_on_sc=False),
    scratch_types=[
        pltpu.VMEM((NCHUNK, CHUNK), jnp.int32),
        pltpu.VMEM((CHUNK, 16), jnp.float32),
        pltpu.VMEM_SHARED((NPAD, 16), jnp.float32),
        pltpu.SemaphoreType.DMA,
    ],
)


def _make_scatter_kernel(f):
    def body(table_hbm, src_hbm, dst_hbm, out_hbm,
             srcv, dstv, rows, zbuf, accum, sem):
        c = lax.axis_index("c")
        s = lax.axis_index("s")
        wid = c * NS + s

        _fill_rows(zbuf, f, 0.0)

        def zloop(i, _):
            pltpu.sync_copy(zbuf, accum.at[pl.ds(s * ROWS_PER_TILE + i * CHUNK, CHUNK)])
            return 0
        lax.fori_loop(0, ROWS_PER_TILE // CHUNK, zloop, 0)

        pltpu.sync_copy(src_hbm.at[wid], srcv)
        pltpu.sync_copy(dst_hbm.at[wid], dstv)
        plsc.subcore_barrier()

        def body_loop(j, _):
            pltpu.async_copy(table_hbm.at[srcv.at[j]], rows, sem).wait()
            pltpu.sync_copy(rows, accum.at[dstv.at[j]], add=True)
            return 0
        lax.fori_loop(0, NCHUNK, body_loop, 0)

        plsc.subcore_barrier()
        pltpu.sync_copy(
            accum.at[pl.ds(s * ROWS_PER_TILE, ROWS_PER_TILE)],
            out_hbm.at[c, pl.ds(s * ROWS_PER_TILE, ROWS_PER_TILE)],
        )

    return pl.kernel(
        body,
        out_type=jax.ShapeDtypeStruct((NC, NPAD, f), jnp.float32),
        mesh=_MESH,
        compiler_params=pltpu.CompilerParams(use_tc_tiling_on_sc=False),
        scratch_types=[
            pltpu.VMEM((NCHUNK, CHUNK), jnp.int32),
            pltpu.VMEM((NCHUNK, CHUNK), jnp.int32),
            pltpu.VMEM((CHUNK, f), jnp.float32),
            pltpu.VMEM((CHUNK, f), jnp.float32),
            pltpu.VMEM_SHARED((NPAD, f), jnp.float32),
            pltpu.SemaphoreType.DMA,
        ],
    )


_scatter_hid = _make_scatter_kernel(HID)
_scatter_f2 = _make_scatter_kernel(F2)


RBLK = 2000
GRID = N // RBLK


def _dinv_from(degp_ref):
    deg = degp_ref[0, :, 0:1] + degp_ref[1, :, 0:1] + 1.0
    return lax.rsqrt(deg)


def _tc1_body(x_ref, w1_ref, degp_ref, g1_ref):
    dinv = _dinv_from(degp_ref)
    h = jnp.dot(x_ref[...], w1_ref[...], preferred_element_type=jnp.float32)
    g1_ref[...] = h * dinv


_tc1 = pl.pallas_call(
    _tc1_body,
    grid=(GRID,),
    in_specs=[
        pl.BlockSpec((RBLK, D_IN), lambda i: (i, 0)),
        pl.BlockSpec((D_IN, HID), lambda i: (0, 0)),
        pl.BlockSpec((NC, RBLK, 16), lambda i: (0, i, 0)),
    ],
    out_specs=pl.BlockSpec((RBLK, HID), lambda i: (i, 0)),
    out_shape=jax.ShapeDtypeStruct((N, HID), jnp.float32),
)


def _tc2_body(p_ref, g1_ref, degp_ref, b1_ref, w2_ref, g2_ref):
    dinv = _dinv_from(degp_ref)
    agg = p_ref[0] + p_ref[1] + g1_ref[...]
    h1 = jnp.maximum(dinv * agg + b1_ref[...], 0.0)
    h2 = jnp.dot(h1, w2_ref[...], preferred_element_type=jnp.float32)
    g2_ref[...] = h2 * dinv


_tc2 = pl.pallas_call(
    _tc2_body,
    grid=(GRID,),
    in_specs=[
        pl.BlockSpec((NC, RBLK, HID), lambda i: (0, i, 0)),
        pl.BlockSpec((RBLK, HID), lambda i: (i, 0)),
        pl.BlockSpec((NC, RBLK, 16), lambda i: (0, i, 0)),
        pl.BlockSpec((1, HID), lambda i: (0, 0)),
        pl.BlockSpec((HID, F2), lambda i: (0, 0)),
    ],
    out_specs=pl.BlockSpec((RBLK, F2), lambda i: (i, 0)),
    out_shape=jax.ShapeDtypeStruct((N, F2), jnp.float32),
)


def _tc3_body(p_ref, g2_ref, degp_ref, b2_ref, out_ref):
    dinv = _dinv_from(degp_ref)
    z = dinv * (p_ref[0] + p_ref[1] + g2_ref[...]) + b2_ref[...]
    z = z[:, :NCLS]
    m = jnp.max(z, axis=1, keepdims=True)
    zs = z - m
    out_ref[...] = zs - jnp.log(jnp.sum(jnp.exp(zs), axis=1, keepdims=True))


_tc3 = pl.pallas_call(
    _tc3_body,
    grid=(GRID,),
    in_specs=[
        pl.BlockSpec((NC, RBLK, F2), lambda i: (0, i, 0)),
        pl.BlockSpec((RBLK, F2), lambda i: (i, 0)),
        pl.BlockSpec((NC, RBLK, 16), lambda i: (0, i, 0)),
        pl.BlockSpec((1, F2), lambda i: (0, 0)),
    ],
    out_specs=pl.BlockSpec((RBLK, NCLS), lambda i: (i, 0)),
    out_shape=jax.ShapeDtypeStruct((N, NCLS), jnp.float32),
)


@jax.jit
def kernel(x, edge_index, W1, b1, W2, b2):
    src = edge_index[0].astype(jnp.int32)
    dst = edge_index[1].astype(jnp.int32)
    src_p = jnp.concatenate([src, jnp.zeros((EPAD - E,), jnp.int32)])
    dst_p = jnp.concatenate([dst, jnp.full((EPAD - E,), NPAD - 1, jnp.int32)])
    src_r = src_p.reshape(NW, NCHUNK, CHUNK)
    dst_r = dst_p.reshape(NW, NCHUNK, CHUNK)

    w2p = jnp.pad(W2, ((0, 0), (0, F2 - NCLS)))
    b1r = b1.reshape(1, HID)
    b2r = jnp.pad(b2, (0, F2 - NCLS)).reshape(1, F2)

    degp = _deg_kernel(dst_r)
    g1 = _tc1(x, W1, degp)
    p1 = _scatter_hid(g1, src_r, dst_r)
    g2 = _tc2(p1, g1, degp, b1r, w2p)
    p2 = _scatter_f2(g2, src_r, dst_r)
    return _tc3(p2, g2, degp, b2r)


# speedup vs baseline: 18.1446x; 1.1690x over previous
import functools

import jax
import jax.numpy as jnp
from jax import lax
from jax.experimental import pallas as pl
from jax.experimental.pallas import tpu as pltpu
from jax.experimental.pallas import tpu_sc as plsc

N = 10000
E = 320000
D_IN = 128
HID = 64
NCLS = 40
F2 = 48

NC = 2
NS = 16
NW = NC * NS

CHUNK = 128
NCHUNK = 80
EPAD = NW * NCHUNK * CHUNK
NPAD = 10240
ROWS_PER_TILE = NPAD // NS

_MESH = plsc.VectorSubcoreMesh(
    core_axis_name="c", subcore_axis_name="s", num_cores=NC, num_subcores=NS
)


def _fill_rows(buf, f, val):
    def body(i, _):
        for k in range(f // 16):
            buf[i, pl.ds(k * 16, 16)] = jnp.full((16,), val, jnp.float32)
        return 0
    lax.fori_loop(0, CHUNK, body, 0)


def _deg_body(dst_hbm, out_hbm, dstv, onesv, accum, sem):
    c = lax.axis_index("c")
    s = lax.axis_index("s")
    wid = c * NS + s

    _fill_rows(onesv, 16, 0.0)

    def zloop(i, _):
        pltpu.sync_copy(onesv, accum.at[pl.ds(s * ROWS_PER_TILE + i * CHUNK, CHUNK)])
        return 0
    lax.fori_loop(0, ROWS_PER_TILE // CHUNK, zloop, 0)

    _fill_rows(onesv, 16, 1.0)
    pltpu.sync_copy(dst_hbm.at[wid], dstv)
    plsc.subcore_barrier()

    def body(j, _):
        pltpu.sync_copy(onesv, accum.at[dstv.at[j]], add=True)
        return 0
    lax.fori_loop(0, NCHUNK, body, 0)

    plsc.subcore_barrier()
    pltpu.sync_copy(
        accum.at[pl.ds(s * ROWS_PER_TILE, ROWS_PER_TILE)],
        out_hbm.at[c, pl.ds(s * ROWS_PER_TILE, ROWS_PER_TILE)],
    )


_deg_kernel = pl.kernel(
    _deg_body,
    out_type=jax.ShapeDtypeStruct((NC, NPAD, 16), jnp.float32),
    mesh=_MESH,
    compiler_params=pltpu.CompilerParams(use_tc_tiling_on_sc=False),
    scratch_types=[
        pltpu.VMEM((NCHUNK, CHUNK), jnp.int32),
        pltpu.VMEM((CHUNK, 16), jnp.float32),
        pltpu.VMEM_SHARED((NPAD, 16), jnp.float32),
        pltpu.SemaphoreType.DMA,
    ],
)


def _make_scatter_kernel(f):
    def body(table_hbm, src_hbm, dst_hbm, out_hbm,
             srcv, dstv, rows0, rows1, zbuf, accum, sem0, sem1):
        c = lax.axis_index("c")
        s = lax.axis_index("s")
        wid = c * NS + s

        _fill_rows(zbuf, f, 0.0)

        def zloop(i, _):
            pltpu.sync_copy(zbuf, accum.at[pl.ds(s * ROWS_PER_TILE + i * CHUNK, CHUNK)])
            return 0
        lax.fori_loop(0, ROWS_PER_TILE // CHUNK, zloop, 0)

        pltpu.sync_copy(src_hbm.at[wid], srcv)
        pltpu.sync_copy(dst_hbm.at[wid], dstv)
        plsc.subcore_barrier()

        pltpu.async_copy(table_hbm.at[srcv.at[0]], rows0, sem0)

        def body_loop(i, _):
            j = 2 * i
            pltpu.async_copy(table_hbm.at[srcv.at[j + 1]], rows1, sem1)
            pltpu.make_async_copy(table_hbm.at[srcv.at[j]], rows0, sem0).wait()
            pltpu.sync_copy(rows0, accum.at[dstv.at[j]], add=True)

            @pl.when(i < NCHUNK // 2 - 1)
            def _():
                pltpu.async_copy(table_hbm.at[srcv.at[j + 2]], rows0, sem0)

            pltpu.make_async_copy(table_hbm.at[srcv.at[j + 1]], rows1, sem1).wait()
            pltpu.sync_copy(rows1, accum.at[dstv.at[j + 1]], add=True)
            return 0
        lax.fori_loop(0, NCHUNK // 2, body_loop, 0)

        plsc.subcore_barrier()
        pltpu.sync_copy(
            accum.at[pl.ds(s * ROWS_PER_TILE, ROWS_PER_TILE)],
            out_hbm.at[c, pl.ds(s * ROWS_PER_TILE, ROWS_PER_TILE)],
        )

    return pl.kernel(
        body,
        out_type=jax.ShapeDtypeStruct((NC, NPAD, f), jnp.float32),
        mesh=_MESH,
        compiler_params=pltpu.CompilerParams(use_tc_tiling_on_sc=False),
        scratch_types=[
            pltpu.VMEM((NCHUNK, CHUNK), jnp.int32),
            pltpu.VMEM((NCHUNK, CHUNK), jnp.int32),
            pltpu.VMEM((CHUNK, f), jnp.float32),
            pltpu.VMEM((CHUNK, f), jnp.float32),
            pltpu.VMEM((CHUNK, f), jnp.float32),
            pltpu.VMEM_SHARED((NPAD, f), jnp.float32),
            pltpu.SemaphoreType.DMA,
            pltpu.SemaphoreType.DMA,
        ],
    )


_scatter_hid = _make_scatter_kernel(HID)
_scatter_f2 = _make_scatter_kernel(F2)


RBLK = 2000
GRID = N // RBLK


def _dinv_from(degp_ref):
    deg = degp_ref[0, :, 0:1] + degp_ref[1, :, 0:1] + 1.0
    return lax.rsqrt(deg)


def _tc1_body(x_ref, w1_ref, degp_ref, g1_ref):
    dinv = _dinv_from(degp_ref)
    h = jnp.dot(x_ref[...], w1_ref[...], preferred_element_type=jnp.float32)
    g1_ref[...] = h * dinv


_tc1 = pl.pallas_call(
    _tc1_body,
    grid=(GRID,),
    in_specs=[
        pl.BlockSpec((RBLK, D_IN), lambda i: (i, 0)),
        pl.BlockSpec((D_IN, HID), lambda i: (0, 0)),
        pl.BlockSpec((NC, RBLK, 16), lambda i: (0, i, 0)),
    ],
    out_specs=pl.BlockSpec((RBLK, HID), lambda i: (i, 0)),
    out_shape=jax.ShapeDtypeStruct((N, HID), jnp.float32),
)


def _tc2_body(p_ref, g1_ref, degp_ref, b1_ref, w2_ref, g2_ref):
    dinv = _dinv_from(degp_ref)
    agg = p_ref[0] + p_ref[1] + g1_ref[...]
    h1 = jnp.maximum(dinv * agg + b1_ref[...], 0.0)
    h2 = jnp.dot(h1, w2_ref[...], preferred_element_type=jnp.float32)
    g2_ref[...] = h2 * dinv


_tc2 = pl.pallas_call(
    _tc2_body,
    grid=(GRID,),
    in_specs=[
        pl.BlockSpec((NC, RBLK, HID), lambda i: (0, i, 0)),
        pl.BlockSpec((RBLK, HID), lambda i: (i, 0)),
        pl.BlockSpec((NC, RBLK, 16), lambda i: (0, i, 0)),
        pl.BlockSpec((1, HID), lambda i: (0, 0)),
        pl.BlockSpec((HID, F2), lambda i: (0, 0)),
    ],
    out_specs=pl.BlockSpec((RBLK, F2), lambda i: (i, 0)),
    out_shape=jax.ShapeDtypeStruct((N, F2), jnp.float32),
)


def _tc3_body(p_ref, g2_ref, degp_ref, b2_ref, out_ref):
    dinv = _dinv_from(degp_ref)
    z = dinv * (p_ref[0] + p_ref[1] + g2_ref[...]) + b2_ref[...]
    z = z[:, :NCLS]
    m = jnp.max(z, axis=1, keepdims=True)
    zs = z - m
    out_ref[...] = zs - jnp.log(jnp.sum(jnp.exp(zs), axis=1, keepdims=True))


_tc3 = pl.pallas_call(
    _tc3_body,
    grid=(GRID,),
    in_specs=[
        pl.BlockSpec((NC, RBLK, F2), lambda i: (0, i, 0)),
        pl.BlockSpec((RBLK, F2), lambda i: (i, 0)),
        pl.BlockSpec((NC, RBLK, 16), lambda i: (0, i, 0)),
        pl.BlockSpec((1, F2), lambda i: (0, 0)),
    ],
    out_specs=pl.BlockSpec((RBLK, NCLS), lambda i: (i, 0)),
    out_shape=jax.ShapeDtypeStruct((N, NCLS), jnp.float32),
)


@jax.jit
def kernel(x, edge_index, W1, b1, W2, b2):
    src = edge_index[0].astype(jnp.int32)
    dst = edge_index[1].astype(jnp.int32)
    pad_dst = N + jnp.arange(EPAD - E, dtype=jnp.int32) % (NPAD - N)
    src_p = jnp.concatenate([src, jnp.zeros((EPAD - E,), jnp.int32)])
    dst_p = jnp.concatenate([dst, pad_dst])
    src_r = src_p.reshape(NW, NCHUNK, CHUNK)
    dst_r = dst_p.reshape(NW, NCHUNK, CHUNK)

    w2p = jnp.pad(W2, ((0, 0), (0, F2 - NCLS)))
    b1r = b1.reshape(1, HID)
    b2r = jnp.pad(b2, (0, F2 - NCLS)).reshape(1, F2)

    degp = _deg_kernel(dst_r)
    g1 = _tc1(x, W1, degp)
    p1 = _scatter_hid(g1, src_r, dst_r)
    g2 = _tc2(p1, g1, degp, b1r, w2p)
    p2 = _scatter_f2(g2, src_r, dst_r)
    return _tc3(p2, g2, degp, b2r)


# speedup vs baseline: 18.2562x; 1.0062x over previous
import functools

import jax
import jax.numpy as jnp
from jax import lax
from jax.experimental import pallas as pl
from jax.experimental.pallas import tpu as pltpu
from jax.experimental.pallas import tpu_sc as plsc

N = 10000
E = 320000
D_IN = 128
HID = 64
NCLS = 40
F2 = 48

NC = 2
NS = 16
NW = NC * NS

CHUNK = 128
NCHUNK = 80
EPAD = NW * NCHUNK * CHUNK
NPAD = 10240
ROWS_PER_TILE = NPAD // NS

_MESH = plsc.VectorSubcoreMesh(
    core_axis_name="c", subcore_axis_name="s", num_cores=NC, num_subcores=NS
)


def _fill_rows(buf, f, val):
    def body(i, _):
        for k in range(f // 16):
            buf[i, pl.ds(k * 16, 16)] = jnp.full((16,), val, jnp.float32)
        return 0
    lax.fori_loop(0, CHUNK, body, 0)


def _deg_body(dst_hbm, out_hbm, dstv, onesv, accum, sem):
    c = lax.axis_index("c")
    s = lax.axis_index("s")
    wid = c * NS + s

    _fill_rows(onesv, 16, 0.0)

    def zloop(i, _):
        pltpu.sync_copy(onesv, accum.at[pl.ds(s * ROWS_PER_TILE + i * CHUNK, CHUNK)])
        return 0
    lax.fori_loop(0, ROWS_PER_TILE // CHUNK, zloop, 0)

    _fill_rows(onesv, 16, 1.0)
    pltpu.sync_copy(dst_hbm.at[wid], dstv)
    plsc.subcore_barrier()

    def body(j, _):
        pltpu.sync_copy(onesv, accum.at[dstv.at[j]], add=True)
        return 0
    lax.fori_loop(0, NCHUNK, body, 0)

    plsc.subcore_barrier()
    pltpu.sync_copy(
        accum.at[pl.ds(s * ROWS_PER_TILE, ROWS_PER_TILE)],
        out_hbm.at[c, pl.ds(s * ROWS_PER_TILE, ROWS_PER_TILE)],
    )


_deg_kernel = pl.kernel(
    _deg_body,
    out_type=jax.ShapeDtypeStruct((NC, NPAD, 16), jnp.float32),
    mesh=_MESH,
    compiler_params=pltpu.CompilerParams(use_tc_tiling_on_sc=False),
    scratch_types=[
        pltpu.VMEM((NCHUNK, CHUNK), jnp.int32),
        pltpu.VMEM((CHUNK, 16), jnp.float32),
        pltpu.VMEM_SHARED((NPAD, 16), jnp.float32),
        pltpu.SemaphoreType.DMA,
    ],
)


NBUF = 4
LOOKAHEAD = 2


def _make_scatter_kernel(f):
    def body(table_hbm, src_hbm, dst_hbm, out_hbm,
             srcv, dstv, rows0, rows1, rows2, rows3, accum,
             gsem0, gsem1, gsem2, gsem3, ssem0, ssem1, ssem2, ssem3,
             isem0, isem1):
        rows = [rows0, rows1, rows2, rows3]
        gsem = [gsem0, gsem1, gsem2, gsem3]
        ssem = [ssem0, ssem1, ssem2, ssem3]
        c = lax.axis_index("c")
        s = lax.axis_index("s")
        wid = c * NS + s

        pltpu.async_copy(src_hbm.at[wid], srcv, isem0)
        pltpu.async_copy(dst_hbm.at[wid], dstv, isem1)

        _fill_rows(rows0, f, 0.0)

        def zloop(i, _):
            pltpu.sync_copy(rows0, accum.at[pl.ds(s * ROWS_PER_TILE + i * CHUNK, CHUNK)])
            return 0
        lax.fori_loop(0, ROWS_PER_TILE // CHUNK, zloop, 0)

        pltpu.make_async_copy(src_hbm.at[wid], srcv, isem0).wait()
        pltpu.make_async_copy(dst_hbm.at[wid], dstv, isem1).wait()
        plsc.subcore_barrier()

        for b in range(LOOKAHEAD):
            pltpu.async_copy(table_hbm.at[srcv.at[b]], rows[b], gsem[b])

        def body_loop(i, _):
            for b in range(NBUF):
                j = i * NBUF + b
                pb = (b + LOOKAHEAD) % NBUF

                @pl.when(j + LOOKAHEAD < NCHUNK)
                def _():
                    @pl.when(j - LOOKAHEAD >= 0)
                    def _():
                        pltpu.make_async_copy(
                            rows[pb], accum.at[dstv.at[j]], ssem[pb]).wait()
                    pltpu.async_copy(
                        table_hbm.at[srcv.at[j + LOOKAHEAD]], rows[pb], gsem[pb])

                pltpu.make_async_copy(table_hbm.at[srcv.at[j]], rows[b], gsem[b]).wait()
                pltpu.async_copy(rows[b], accum.at[dstv.at[j]], ssem[b], add=True)
            return 0
        lax.fori_loop(0, NCHUNK // NBUF, body_loop, 0)

        for b in range(NBUF):
            pltpu.make_async_copy(rows[b], accum.at[dstv.at[0]], ssem[b]).wait()

        plsc.subcore_barrier()
        pltpu.sync_copy(
            accum.at[pl.ds(s * ROWS_PER_TILE, ROWS_PER_TILE)],
            out_hbm.at[c, pl.ds(s * ROWS_PER_TILE, ROWS_PER_TILE)],
        )

    return pl.kernel(
        body,
        out_type=jax.ShapeDtypeStruct((NC, NPAD, f), jnp.float32),
        mesh=_MESH,
        compiler_params=pltpu.CompilerParams(use_tc_tiling_on_sc=False),
        scratch_types=[
            pltpu.VMEM((NCHUNK, CHUNK), jnp.int32),
            pltpu.VMEM((NCHUNK, CHUNK), jnp.int32),
            pltpu.VMEM((CHUNK, f), jnp.float32),
            pltpu.VMEM((CHUNK, f), jnp.float32),
            pltpu.VMEM((CHUNK, f), jnp.float32),
            pltpu.VMEM((CHUNK, f), jnp.float32),
            pltpu.VMEM_SHARED((NPAD, f), jnp.float32),
        ] + [pltpu.SemaphoreType.DMA] * 10,
    )


_scatter_hid = _make_scatter_kernel(HID)
_scatter_f2 = _make_scatter_kernel(F2)


RBLK = 2000
GRID = N // RBLK


def _dinv_from(degp_ref):
    deg = degp_ref[0, :, 0:1] + degp_ref[1, :, 0:1] + 1.0
    return lax.rsqrt(deg)


def _tc1_body(x_ref, w1_ref, degp_ref, g1_ref):
    dinv = _dinv_from(degp_ref)
    h = jnp.dot(x_ref[...], w1_ref[...], preferred_element_type=jnp.float32)
    g1_ref[...] = h * dinv


_tc1 = pl.pallas_call(
    _tc1_body,
    grid=(GRID,),
    in_specs=[
        pl.BlockSpec((RBLK, D_IN), lambda i: (i, 0)),
        pl.BlockSpec((D_IN, HID), lambda i: (0, 0)),
        pl.BlockSpec((NC, RBLK, 16), lambda i: (0, i, 0)),
    ],
    out_specs=pl.BlockSpec((RBLK, HID), lambda i: (i, 0)),
    out_shape=jax.ShapeDtypeStruct((N, HID), jnp.float32),
)


def _tc2_body(p_ref, g1_ref, degp_ref, b1_ref, w2_ref, g2_ref):
    dinv = _dinv_from(degp_ref)
    agg = p_ref[0] + p_ref[1] + g1_ref[...]
    h1 = jnp.maximum(dinv * agg + b1_ref[...], 0.0)
    h2 = jnp.dot(h1, w2_ref[...], preferred_element_type=jnp.float32)
    g2_ref[...] = h2 * dinv


_tc2 = pl.pallas_call(
    _tc2_body,
    grid=(GRID,),
    in_specs=[
        pl.BlockSpec((NC, RBLK, HID), lambda i: (0, i, 0)),
        pl.BlockSpec((RBLK, HID), lambda i: (i, 0)),
        pl.BlockSpec((NC, RBLK, 16), lambda i: (0, i, 0)),
        pl.BlockSpec((1, HID), lambda i: (0, 0)),
        pl.BlockSpec((HID, F2), lambda i: (0, 0)),
    ],
    out_specs=pl.BlockSpec((RBLK, F2), lambda i: (i, 0)),
    out_shape=jax.ShapeDtypeStruct((N, F2), jnp.float32),
)


def _tc3_body(p_ref, g2_ref, degp_ref, b2_ref, out_ref):
    dinv = _dinv_from(degp_ref)
    z = dinv * (p_ref[0] + p_ref[1] + g2_ref[...]) + b2_ref[...]
    z = z[:, :NCLS]
    m = jnp.max(z, axis=1, keepdims=True)
    zs = z - m
    out_ref[...] = zs - jnp.log(jnp.sum(jnp.exp(zs), axis=1, keepdims=True))


_tc3 = pl.pallas_call(
    _tc3_body,
    grid=(GRID,),
    in_specs=[
        pl.BlockSpec((NC, RBLK, F2), lambda i: (0, i, 0)),
        pl.BlockSpec((RBLK, F2), lambda i: (i, 0)),
        pl.BlockSpec((NC, RBLK, 16), lambda i: (0, i, 0)),
        pl.BlockSpec((1, F2), lambda i: (0, 0)),
    ],
    out_specs=pl.BlockSpec((RBLK, NCLS), lambda i: (i, 0)),
    out_shape=jax.ShapeDtypeStruct((N, NCLS), jnp.float32),
)


@jax.jit
def kernel(x, edge_index, W1, b1, W2, b2):
    src = edge_index[0].astype(jnp.int32)
    dst = edge_index[1].astype(jnp.int32)
    pad_dst = N + jnp.arange(EPAD - E, dtype=jnp.int32) % (NPAD - N)
    src_p = jnp.concatenate([src, jnp.zeros((EPAD - E,), jnp.int32)])
    dst_p = jnp.concatenate([dst, pad_dst])
    src_r = src_p.reshape(NW, NCHUNK, CHUNK)
    dst_r = dst_p.reshape(NW, NCHUNK, CHUNK)

    w2p = jnp.pad(W2, ((0, 0), (0, F2 - NCLS)))
    b1r = b1.reshape(1, HID)
    b2r = jnp.pad(b2, (0, F2 - NCLS)).reshape(1, F2)

    degp = _deg_kernel(dst_r)
    g1 = _tc1(x, W1, degp)
    p1 = _scatter_hid(g1, src_r, dst_r)
    g2 = _tc2(p1, g1, degp, b1r, w2p)
    p2 = _scatter_f2(g2, src_r, dst_r)
    return _tc3(p2, g2, degp, b2r)


# speedup vs baseline: 24.6450x; 1.3500x over previous
import functools

import jax
import jax.numpy as jnp
from jax import lax
from jax.experimental import pallas as pl
from jax.experimental.pallas import tpu as pltpu
from jax.experimental.pallas import tpu_sc as plsc

N = 10000
E = 320000
D_IN = 128
HID = 64
NCLS = 40
F2 = 48

NC = 2
NS = 16
NW = NC * NS

CHUNK = 128
NCHUNK = 80
EPAD = NW * NCHUNK * CHUNK
NPAD = 10240
ROWS_PER_TILE = NPAD // NS

_MESH = plsc.VectorSubcoreMesh(
    core_axis_name="c", subcore_axis_name="s", num_cores=NC, num_subcores=NS
)


def _fill_rows(buf, f, val):
    def body(i, _):
        for k in range(f // 16):
            buf[i, pl.ds(k * 16, 16)] = jnp.full((16,), val, jnp.float32)
        return 0
    lax.fori_loop(0, CHUNK, body, 0)


def _deg_body(dst_hbm, out_hbm, dstv, onesv, accum, sem):
    c = lax.axis_index("c")
    s = lax.axis_index("s")
    wid = c * NS + s

    _fill_rows(onesv, 16, 0.0)

    def zloop(i, _):
        pltpu.sync_copy(onesv, accum.at[pl.ds(s * ROWS_PER_TILE + i * CHUNK, CHUNK)])
        return 0
    lax.fori_loop(0, ROWS_PER_TILE // CHUNK, zloop, 0)

    _fill_rows(onesv, 16, 1.0)
    pltpu.sync_copy(dst_hbm.at[wid], dstv)
    plsc.subcore_barrier()

    def body(j, _):
        pltpu.sync_copy(onesv, accum.at[dstv.at[j]], add=True)
        return 0
    lax.fori_loop(0, NCHUNK, body, 0)

    plsc.subcore_barrier()
    pltpu.sync_copy(
        accum.at[pl.ds(s * ROWS_PER_TILE, ROWS_PER_TILE)],
        out_hbm.at[c, pl.ds(s * ROWS_PER_TILE, ROWS_PER_TILE)],
    )


_deg_kernel = pl.kernel(
    _deg_body,
    out_type=jax.ShapeDtypeStruct((NC, NPAD, 16), jnp.float32),
    mesh=_MESH,
    compiler_params=pltpu.CompilerParams(use_tc_tiling_on_sc=False),
    scratch_types=[
        pltpu.VMEM((NCHUNK, CHUNK), jnp.int32),
        pltpu.VMEM((CHUNK, 16), jnp.float32),
        pltpu.VMEM_SHARED((NPAD, 16), jnp.float32),
        pltpu.SemaphoreType.DMA,
    ],
)


NBUF = 4
LOOKAHEAD = 2


def _make_scatter_kernel(f, split):
    use_tbl = split > 0

    def body(table_hbm, src_hbm, dst_hbm, out_hbm, *scr):
        srcv, dstv = scr[0], scr[1]
        rows = list(scr[2:6])
        accum = scr[6]
        idx = 7
        if use_tbl:
            tbl = scr[7]
            idx = 8
        gsem = list(scr[idx:idx + 4])
        ssem = list(scr[idx + 4:idx + 8])
        isem0, isem1 = scr[idx + 8], scr[idx + 9]
        if use_tbl:
            tsem = scr[idx + 10]
        c = lax.axis_index("c")
        s = lax.axis_index("s")
        wid = c * NS + s
        tsl = pl.ds(s * ROWS_PER_TILE, ROWS_PER_TILE)

        def src_of(b):
            return tbl if (use_tbl and b < split) else table_hbm

        if use_tbl:
            pltpu.async_copy(table_hbm.at[tsl], tbl.at[tsl], tsem)
        pltpu.async_copy(src_hbm.at[wid], srcv, isem0)
        pltpu.async_copy(dst_hbm.at[wid], dstv, isem1)

        _fill_rows(rows[0], f, 0.0)

        def zloop(i, _):
            pltpu.sync_copy(rows[0], accum.at[pl.ds(s * ROWS_PER_TILE + i * CHUNK, CHUNK)])
            return 0
        lax.fori_loop(0, ROWS_PER_TILE // CHUNK, zloop, 0)

        if use_tbl:
            pltpu.make_async_copy(table_hbm.at[tsl], tbl.at[tsl], tsem).wait()
        pltpu.make_async_copy(src_hbm.at[wid], srcv, isem0).wait()
        pltpu.make_async_copy(dst_hbm.at[wid], dstv, isem1).wait()
        plsc.subcore_barrier()

        for b in range(LOOKAHEAD):
            pltpu.async_copy(src_of(b).at[srcv.at[b]], rows[b], gsem[b])

        def body_loop(i, _):
            for b in range(NBUF):
                j = i * NBUF + b
                pb = (b + LOOKAHEAD) % NBUF

                @pl.when(j + LOOKAHEAD < NCHUNK)
                def _():
                    @pl.when(j - LOOKAHEAD >= 0)
                    def _():
                        pltpu.make_async_copy(
                            rows[pb], accum.at[dstv.at[j]], ssem[pb]).wait()
                    pltpu.async_copy(
                        src_of(pb).at[srcv.at[j + LOOKAHEAD]], rows[pb], gsem[pb])

                pltpu.make_async_copy(src_of(b).at[srcv.at[j]], rows[b], gsem[b]).wait()
                pltpu.async_copy(rows[b], accum.at[dstv.at[j]], ssem[b], add=True)
            return 0
        lax.fori_loop(0, NCHUNK // NBUF, body_loop, 0)

        for b in range(NBUF):
            pltpu.make_async_copy(rows[b], accum.at[dstv.at[0]], ssem[b]).wait()

        plsc.subcore_barrier()
        pltpu.sync_copy(
            accum.at[pl.ds(s * ROWS_PER_TILE, ROWS_PER_TILE)],
            out_hbm.at[c, pl.ds(s * ROWS_PER_TILE, ROWS_PER_TILE)],
        )

    scratch = [
        pltpu.VMEM((NCHUNK, CHUNK), jnp.int32),
        pltpu.VMEM((NCHUNK, CHUNK), jnp.int32),
        pltpu.VMEM((CHUNK, f), jnp.float32),
        pltpu.VMEM((CHUNK, f), jnp.float32),
        pltpu.VMEM((CHUNK, f), jnp.float32),
        pltpu.VMEM((CHUNK, f), jnp.float32),
        pltpu.VMEM_SHARED((NPAD, f), jnp.float32),
    ]
    if use_tbl:
        scratch.append(pltpu.VMEM_SHARED((NPAD, f), jnp.float32))
    nsem = 11 if use_tbl else 10
    return pl.kernel(
        body,
        out_type=jax.ShapeDtypeStruct((NC, NPAD, f), jnp.float32),
        mesh=_MESH,
        compiler_params=pltpu.CompilerParams(use_tc_tiling_on_sc=False),
        scratch_types=scratch + [pltpu.SemaphoreType.DMA] * nsem,
    )


_scatter_hid = _make_scatter_kernel(HID, 0)
_scatter_f2 = _make_scatter_kernel(F2, 4)


RBLK = 2000
GRID = N // RBLK


def _dinv_from(degp_ref):
    deg = degp_ref[0, :, 0:1] + degp_ref[1, :, 0:1] + 1.0
    return lax.rsqrt(deg)


def _tc1_body(x_ref, w1_ref, degp_ref, g1_ref):
    dinv = _dinv_from(degp_ref)
    h = jnp.dot(x_ref[...], w1_ref[...], preferred_element_type=jnp.float32)
    g1_ref[...] = h * dinv


_tc1 = pl.pallas_call(
    _tc1_body,
    grid=(GRID,),
    in_specs=[
        pl.BlockSpec((RBLK, D_IN), lambda i: (i, 0)),
        pl.BlockSpec((D_IN, HID), lambda i: (0, 0)),
        pl.BlockSpec((NC, RBLK, 16), lambda i: (0, i, 0)),
    ],
    out_specs=pl.BlockSpec((RBLK, HID), lambda i: (i, 0)),
    out_shape=jax.ShapeDtypeStruct((N, HID), jnp.float32),
)


def _tc2_body(p_ref, g1_ref, degp_ref, b1_ref, w2_ref, g2_ref):
    dinv = _dinv_from(degp_ref)
    agg = p_ref[0] + p_ref[1] + g1_ref[...]
    h1 = jnp.maximum(dinv * agg + b1_ref[...], 0.0)
    h2 = jnp.dot(h1, w2_ref[...], preferred_element_type=jnp.float32)
    g2_ref[...] = h2 * dinv


_tc2 = pl.pallas_call(
    _tc2_body,
    grid=(GRID,),
    in_specs=[
        pl.BlockSpec((NC, RBLK, HID), lambda i: (0, i, 0)),
        pl.BlockSpec((RBLK, HID), lambda i: (i, 0)),
        pl.BlockSpec((NC, RBLK, 16), lambda i: (0, i, 0)),
        pl.BlockSpec((1, HID), lambda i: (0, 0)),
        pl.BlockSpec((HID, F2), lambda i: (0, 0)),
    ],
    out_specs=pl.BlockSpec((RBLK, F2), lambda i: (i, 0)),
    out_shape=jax.ShapeDtypeStruct((N, F2), jnp.float32),
)


def _tc3_body(p_ref, g2_ref, degp_ref, b2_ref, out_ref):
    dinv = _dinv_from(degp_ref)
    z = dinv * (p_ref[0] + p_ref[1] + g2_ref[...]) + b2_ref[...]
    z = z[:, :NCLS]
    m = jnp.max(z, axis=1, keepdims=True)
    zs = z - m
    out_ref[...] = zs - jnp.log(jnp.sum(jnp.exp(zs), axis=1, keepdims=True))


_tc3 = pl.pallas_call(
    _tc3_body,
    grid=(GRID,),
    in_specs=[
        pl.BlockSpec((NC, RBLK, F2), lambda i: (0, i, 0)),
        pl.BlockSpec((RBLK, F2), lambda i: (i, 0)),
        pl.BlockSpec((NC, RBLK, 16), lambda i: (0, i, 0)),
        pl.BlockSpec((1, F2), lambda i: (0, 0)),
    ],
    out_specs=pl.BlockSpec((RBLK, NCLS), lambda i: (i, 0)),
    out_shape=jax.ShapeDtypeStruct((N, NCLS), jnp.float32),
)


@jax.jit
def kernel(x, edge_index, W1, b1, W2, b2):
    src = edge_index[0].astype(jnp.int32)
    dst = edge_index[1].astype(jnp.int32)
    pad_dst = N + jnp.arange(EPAD - E, dtype=jnp.int32) % (NPAD - N)
    src_p = jnp.concatenate([src, jnp.zeros((EPAD - E,), jnp.int32)])
    dst_p = jnp.concatenate([dst, pad_dst])
    src_r = src_p.reshape(NW, NCHUNK, CHUNK)
    dst_r = dst_p.reshape(NW, NCHUNK, CHUNK)

    w2p = jnp.pad(W2, ((0, 0), (0, F2 - NCLS)))
    b1r = b1.reshape(1, HID)
    b2r = jnp.pad(b2, (0, F2 - NCLS)).reshape(1, F2)

    degp = _deg_kernel(dst_r)
    g1 = _tc1(x, W1, degp)
    p1 = _scatter_hid(jnp.pad(g1, ((0, NPAD - N), (0, 0))), src_r, dst_r)
    g2 = _tc2(p1, g1, degp, b1r, w2p)
    p2 = _scatter_f2(jnp.pad(g2, ((0, NPAD - N), (0, 0))), src_r, dst_r)
    return _tc3(p2, g2, degp, b2r)


# speedup vs baseline: 39.1453x; 1.5884x over previous
import functools

import jax
import jax.numpy as jnp
from jax import lax
from jax.experimental import pallas as pl
from jax.experimental.pallas import tpu as pltpu
from jax.experimental.pallas import tpu_sc as plsc

N = 10000
E = 320000
D_IN = 128
HID = 64
NCLS = 40
F2 = 48

NC = 2
NS = 16
NW = NC * NS

CHUNK = 128
NCHUNK = 80
EPAD = NW * NCHUNK * CHUNK
NPAD = 10240
ROWS_PER_TILE = NPAD // NS

_MESH = plsc.VectorSubcoreMesh(
    core_axis_name="c", subcore_axis_name="s", num_cores=NC, num_subcores=NS
)


def _fill_rows(buf, f, val):
    def body(i, _):
        for k in range(f // 16):
            buf[i, pl.ds(k * 16, 16)] = jnp.full((16,), val, jnp.float32)
        return 0
    lax.fori_loop(0, CHUNK, body, 0)


def _deg_body(dst_hbm, out_hbm, dstv, onesv, accum, sem):
    c = lax.axis_index("c")
    s = lax.axis_index("s")
    wid = c * NS + s

    _fill_rows(onesv, 16, 0.0)

    def zloop(i, _):
        pltpu.sync_copy(onesv, accum.at[pl.ds(s * ROWS_PER_TILE + i * CHUNK, CHUNK)])
        return 0
    lax.fori_loop(0, ROWS_PER_TILE // CHUNK, zloop, 0)

    _fill_rows(onesv, 16, 1.0)
    pltpu.sync_copy(dst_hbm.at[wid], dstv)
    plsc.subcore_barrier()

    def body(j, _):
        pltpu.sync_copy(onesv, accum.at[dstv.at[j]], add=True)
        return 0
    lax.fori_loop(0, NCHUNK, body, 0)

    plsc.subcore_barrier()
    pltpu.sync_copy(
        accum.at[pl.ds(s * ROWS_PER_TILE, ROWS_PER_TILE)],
        out_hbm.at[c, pl.ds(s * ROWS_PER_TILE, ROWS_PER_TILE)],
    )


_deg_kernel = pl.kernel(
    _deg_body,
    out_type=jax.ShapeDtypeStruct((NC, NPAD, 16), jnp.float32),
    mesh=_MESH,
    compiler_params=pltpu.CompilerParams(use_tc_tiling_on_sc=False),
    scratch_types=[
        pltpu.VMEM((NCHUNK, CHUNK), jnp.int32),
        pltpu.VMEM((CHUNK, 16), jnp.float32),
        pltpu.VMEM_SHARED((NPAD, 16), jnp.float32),
        pltpu.SemaphoreType.DMA,
    ],
)


def _make_scatter_kernel(f, split, nbuf=4, lookahead=2):
    use_tbl = split > 0
    NBUF, LOOKAHEAD = nbuf, lookahead

    def body(table_hbm, src_hbm, dst_hbm, out_hbm, *scr):
        srcv, dstv = scr[0], scr[1]
        rows = list(scr[2:2 + NBUF])
        accum = scr[2 + NBUF]
        idx = 3 + NBUF
        if use_tbl:
            tbl = scr[idx]
            idx += 1
        gsem = list(scr[idx:idx + NBUF])
        ssem = list(scr[idx + NBUF:idx + 2 * NBUF])
        isem0, isem1 = scr[idx + 2 * NBUF], scr[idx + 2 * NBUF + 1]
        if use_tbl:
            tsem = scr[idx + 2 * NBUF + 2]
        c = lax.axis_index("c")
        s = lax.axis_index("s")
        wid = c * NS + s
        tsl = pl.ds(s * ROWS_PER_TILE, ROWS_PER_TILE)

        def src_of(b):
            return tbl if (use_tbl and b < split) else table_hbm

        if use_tbl:
            pltpu.async_copy(table_hbm.at[tsl], tbl.at[tsl], tsem)
        pltpu.async_copy(src_hbm.at[wid], srcv, isem0)
        pltpu.async_copy(dst_hbm.at[wid], dstv, isem1)

        _fill_rows(rows[0], f, 0.0)

        def zloop(i, _):
            pltpu.sync_copy(rows[0], accum.at[pl.ds(s * ROWS_PER_TILE + i * CHUNK, CHUNK)])
            return 0
        lax.fori_loop(0, ROWS_PER_TILE // CHUNK, zloop, 0)

        if use_tbl:
            pltpu.make_async_copy(table_hbm.at[tsl], tbl.at[tsl], tsem).wait()
        pltpu.make_async_copy(src_hbm.at[wid], srcv, isem0).wait()
        pltpu.make_async_copy(dst_hbm.at[wid], dstv, isem1).wait()
        plsc.subcore_barrier()

        for b in range(LOOKAHEAD):
            pltpu.async_copy(src_of(b).at[srcv.at[b]], rows[b], gsem[b])

        def body_loop(i, _):
            for b in range(NBUF):
                j = i * NBUF + b
                pb = (b + LOOKAHEAD) % NBUF

                @pl.when(j + LOOKAHEAD < NCHUNK)
                def _():
                    @pl.when(j - LOOKAHEAD >= 0)
                    def _():
                        pltpu.make_async_copy(
                            rows[pb], accum.at[dstv.at[j]], ssem[pb]).wait()
                    pltpu.async_copy(
                        src_of(pb).at[srcv.at[j + LOOKAHEAD]], rows[pb], gsem[pb])

                pltpu.make_async_copy(src_of(b).at[srcv.at[j]], rows[b], gsem[b]).wait()
                pltpu.async_copy(rows[b], accum.at[dstv.at[j]], ssem[b], add=True)
            return 0
        lax.fori_loop(0, NCHUNK // NBUF, body_loop, 0)

        for b in range(NBUF):
            pltpu.make_async_copy(rows[b], accum.at[dstv.at[0]], ssem[b]).wait()

        plsc.subcore_barrier()
        pltpu.sync_copy(
            accum.at[pl.ds(s * ROWS_PER_TILE, ROWS_PER_TILE)],
            out_hbm.at[c, pl.ds(s * ROWS_PER_TILE, ROWS_PER_TILE)],
        )

    scratch = [
        pltpu.VMEM((NCHUNK, CHUNK), jnp.int32),
        pltpu.VMEM((NCHUNK, CHUNK), jnp.int32),
    ] + [
        pltpu.VMEM((CHUNK, f), jnp.float32) for _ in range(NBUF)
    ] + [
        pltpu.VMEM_SHARED((NPAD, f), jnp.float32),
    ]
    if use_tbl:
        scratch.append(pltpu.VMEM_SHARED((NPAD, f), jnp.float32))
    nsem = 2 * NBUF + 2 + (1 if use_tbl else 0)
    return pl.kernel(
        body,
        out_type=jax.ShapeDtypeStruct((NC, NPAD, f), jnp.float32),
        mesh=_MESH,
        compiler_params=pltpu.CompilerParams(use_tc_tiling_on_sc=False),
        scratch_types=scratch + [pltpu.SemaphoreType.DMA] * nsem,
    )


_scatter_hid = _make_scatter_kernel(HID, 2, nbuf=2, lookahead=1)
_scatter_f2 = _make_scatter_kernel(F2, 4)


RBLK = 2000
GRID = N // RBLK


def _dinv_from(degp_ref):
    deg = degp_ref[0, :, 0:1] + degp_ref[1, :, 0:1] + 1.0
    return lax.rsqrt(deg)


def _tc1_body(x_ref, w1_ref, degp_ref, g1_ref):
    dinv = _dinv_from(degp_ref)
    h = jnp.dot(x_ref[...], w1_ref[...], preferred_element_type=jnp.float32)
    g1_ref[...] = h * dinv


_tc1 = pl.pallas_call(
    _tc1_body,
    grid=(GRID,),
    in_specs=[
        pl.BlockSpec((RBLK, D_IN), lambda i: (i, 0)),
        pl.BlockSpec((D_IN, HID), lambda i: (0, 0)),
        pl.BlockSpec((NC, RBLK, 16), lambda i: (0, i, 0)),
    ],
    out_specs=pl.BlockSpec((RBLK, HID), lambda i: (i, 0)),
    out_shape=jax.ShapeDtypeStruct((N, HID), jnp.float32),
)


def _tc2_body(p_ref, g1_ref, degp_ref, b1_ref, w2_ref, g2_ref):
    dinv = _dinv_from(degp_ref)
    agg = p_ref[0] + p_ref[1] + g1_ref[...]
    h1 = jnp.maximum(dinv * agg + b1_ref[...], 0.0)
    h2 = jnp.dot(h1, w2_ref[...], preferred_element_type=jnp.float32)
    g2_ref[...] = h2 * dinv


_tc2 = pl.pallas_call(
    _tc2_body,
    grid=(GRID,),
    in_specs=[
        pl.BlockSpec((NC, RBLK, HID), lambda i: (0, i, 0)),
        pl.BlockSpec((RBLK, HID), lambda i: (i, 0)),
        pl.BlockSpec((NC, RBLK, 16), lambda i: (0, i, 0)),
        pl.BlockSpec((1, HID), lambda i: (0, 0)),
        pl.BlockSpec((HID, F2), lambda i: (0, 0)),
    ],
    out_specs=pl.BlockSpec((RBLK, F2), lambda i: (i, 0)),
    out_shape=jax.ShapeDtypeStruct((N, F2), jnp.float32),
)


def _tc3_body(p_ref, g2_ref, degp_ref, b2_ref, out_ref):
    dinv = _dinv_from(degp_ref)
    z = dinv * (p_ref[0] + p_ref[1] + g2_ref[...]) + b2_ref[...]
    z = z[:, :NCLS]
    m = jnp.max(z, axis=1, keepdims=True)
    zs = z - m
    out_ref[...] = zs - jnp.log(jnp.sum(jnp.exp(zs), axis=1, keepdims=True))


_tc3 = pl.pallas_call(
    _tc3_body,
    grid=(GRID,),
    in_specs=[
        pl.BlockSpec((NC, RBLK, F2), lambda i: (0, i, 0)),
        pl.BlockSpec((RBLK, F2), lambda i: (i, 0)),
        pl.BlockSpec((NC, RBLK, 16), lambda i: (0, i, 0)),
        pl.BlockSpec((1, F2), lambda i: (0, 0)),
    ],
    out_specs=pl.BlockSpec((RBLK, NCLS), lambda i: (i, 0)),
    out_shape=jax.ShapeDtypeStruct((N, NCLS), jnp.float32),
)


@jax.jit
def kernel(x, edge_index, W1, b1, W2, b2):
    src = edge_index[0].astype(jnp.int32)
    dst = edge_index[1].astype(jnp.int32)
    pad_dst = N + jnp.arange(EPAD - E, dtype=jnp.int32) % (NPAD - N)
    src_p = jnp.concatenate([src, jnp.zeros((EPAD - E,), jnp.int32)])
    dst_p = jnp.concatenate([dst, pad_dst])
    src_r = src_p.reshape(NW, NCHUNK, CHUNK)
    dst_r = dst_p.reshape(NW, NCHUNK, CHUNK)

    w2p = jnp.pad(W2, ((0, 0), (0, F2 - NCLS)))
    b1r = b1.reshape(1, HID)
    b2r = jnp.pad(b2, (0, F2 - NCLS)).reshape(1, F2)

    degp = _deg_kernel(dst_r)
    g1 = _tc1(x, W1, degp)
    p1 = _scatter_hid(jnp.pad(g1, ((0, NPAD - N), (0, 0))), src_r, dst_r)
    g2 = _tc2(p1, g1, degp, b1r, w2p)
    p2 = _scatter_f2(jnp.pad(g2, ((0, NPAD - N), (0, 0))), src_r, dst_r)
    return _tc3(p2, g2, degp, b2r)


# speedup vs baseline: 39.1987x; 1.0014x over previous
import functools

import jax
import jax.numpy as jnp
from jax import lax
from jax.experimental import pallas as pl
from jax.experimental.pallas import tpu as pltpu
from jax.experimental.pallas import tpu_sc as plsc

N = 10000
E = 320000
D_IN = 128
HID = 64
NCLS = 40
F2 = 48

NC = 2
NS = 16
NW = NC * NS

CHUNK = 128
NCHUNK = 80
EPAD = NW * NCHUNK * CHUNK
NPAD = 10240
ROWS_PER_TILE = NPAD // NS

_MESH = plsc.VectorSubcoreMesh(
    core_axis_name="c", subcore_axis_name="s", num_cores=NC, num_subcores=NS
)


def _fill_rows(buf, f, val):
    def body(i, _):
        for k in range(f // 16):
            buf[i, pl.ds(k * 16, 16)] = jnp.full((16,), val, jnp.float32)
        return 0
    lax.fori_loop(0, CHUNK, body, 0)


def _deg_body(dst_hbm, out_hbm, dstv, onesv, accum, sem):
    c = lax.axis_index("c")
    s = lax.axis_index("s")
    wid = c * NS + s

    _fill_rows(onesv, 16, 0.0)

    def zloop(i, _):
        pltpu.sync_copy(onesv, accum.at[pl.ds(s * ROWS_PER_TILE + i * CHUNK, CHUNK)])
        return 0
    lax.fori_loop(0, ROWS_PER_TILE // CHUNK, zloop, 0)

    _fill_rows(onesv, 16, 1.0)
    pltpu.sync_copy(dst_hbm.at[wid], dstv)
    plsc.subcore_barrier()

    def body(j, _):
        pltpu.sync_copy(onesv, accum.at[dstv.at[j]], add=True)
        return 0
    lax.fori_loop(0, NCHUNK, body, 0)

    plsc.subcore_barrier()
    pltpu.sync_copy(
        accum.at[pl.ds(s * ROWS_PER_TILE, ROWS_PER_TILE)],
        out_hbm.at[c, pl.ds(s * ROWS_PER_TILE, ROWS_PER_TILE)],
    )


_deg_kernel = pl.kernel(
    _deg_body,
    out_type=jax.ShapeDtypeStruct((NC, NPAD, 16), jnp.float32),
    mesh=_MESH,
    compiler_params=pltpu.CompilerParams(use_tc_tiling_on_sc=False),
    scratch_types=[
        pltpu.VMEM((NCHUNK, CHUNK), jnp.int32),
        pltpu.VMEM((CHUNK, 16), jnp.float32),
        pltpu.VMEM_SHARED((NPAD, 16), jnp.float32),
        pltpu.SemaphoreType.DMA,
    ],
)


def _make_scatter_kernel(f, split, nbuf=4, lookahead=2):
    use_tbl = split > 0
    NBUF, LOOKAHEAD = nbuf, lookahead

    def body(table_hbm, src_hbm, dst_hbm, out_hbm, *scr):
        srcv, dstv = scr[0], scr[1]
        rows = list(scr[2:2 + NBUF])
        accum = scr[2 + NBUF]
        idx = 3 + NBUF
        if use_tbl:
            tbl = scr[idx]
            idx += 1
        gsem = list(scr[idx:idx + NBUF])
        ssem = list(scr[idx + NBUF:idx + 2 * NBUF])
        isem0, isem1 = scr[idx + 2 * NBUF], scr[idx + 2 * NBUF + 1]
        if use_tbl:
            tsem = scr[idx + 2 * NBUF + 2]
        c = lax.axis_index("c")
        s = lax.axis_index("s")
        wid = c * NS + s
        tsl = pl.ds(s * (N // NS), N // NS)

        def src_of(b):
            return tbl if (use_tbl and b < split) else table_hbm

        if use_tbl:
            pltpu.async_copy(table_hbm.at[tsl], tbl.at[tsl], tsem)
        pltpu.async_copy(src_hbm.at[wid], srcv, isem0)
        pltpu.async_copy(dst_hbm.at[wid], dstv, isem1)

        _fill_rows(rows[0], f, 0.0)

        def zloop(i, _):
            pltpu.sync_copy(rows[0], accum.at[pl.ds(s * ROWS_PER_TILE + i * CHUNK, CHUNK)])
            return 0
        lax.fori_loop(0, ROWS_PER_TILE // CHUNK, zloop, 0)

        if use_tbl:
            pltpu.make_async_copy(table_hbm.at[tsl], tbl.at[tsl], tsem).wait()
        pltpu.make_async_copy(src_hbm.at[wid], srcv, isem0).wait()
        pltpu.make_async_copy(dst_hbm.at[wid], dstv, isem1).wait()
        plsc.subcore_barrier()

        for b in range(LOOKAHEAD):
            pltpu.async_copy(src_of(b).at[srcv.at[b]], rows[b], gsem[b])

        def body_loop(i, _):
            for b in range(NBUF):
                j = i * NBUF + b
                pb = (b + LOOKAHEAD) % NBUF

                @pl.when(j + LOOKAHEAD < NCHUNK)
                def _():
                    @pl.when(j - LOOKAHEAD >= 0)
                    def _():
                        pltpu.make_async_copy(
                            rows[pb], accum.at[dstv.at[j]], ssem[pb]).wait()
                    pltpu.async_copy(
                        src_of(pb).at[srcv.at[j + LOOKAHEAD]], rows[pb], gsem[pb])

                pltpu.make_async_copy(src_of(b).at[srcv.at[j]], rows[b], gsem[b]).wait()
                pltpu.async_copy(rows[b], accum.at[dstv.at[j]], ssem[b], add=True)
            return 0
        lax.fori_loop(0, NCHUNK // NBUF, body_loop, 0)

        for b in range(NBUF):
            pltpu.make_async_copy(rows[b], accum.at[dstv.at[0]], ssem[b]).wait()

        plsc.subcore_barrier()
        pltpu.sync_copy(
            accum.at[pl.ds(s * ROWS_PER_TILE, ROWS_PER_TILE)],
            out_hbm.at[c, pl.ds(s * ROWS_PER_TILE, ROWS_PER_TILE)],
        )

    scratch = [
        pltpu.VMEM((NCHUNK, CHUNK), jnp.int32),
        pltpu.VMEM((NCHUNK, CHUNK), jnp.int32),
    ] + [
        pltpu.VMEM((CHUNK, f), jnp.float32) for _ in range(NBUF)
    ] + [
        pltpu.VMEM_SHARED((NPAD, f), jnp.float32),
    ]
    if use_tbl:
        scratch.append(pltpu.VMEM_SHARED((N, f), jnp.float32))
    nsem = 2 * NBUF + 2 + (1 if use_tbl else 0)
    return pl.kernel(
        body,
        out_type=jax.ShapeDtypeStruct((NC, NPAD, f), jnp.float32),
        mesh=_MESH,
        compiler_params=pltpu.CompilerParams(use_tc_tiling_on_sc=False),
        scratch_types=scratch + [pltpu.SemaphoreType.DMA] * nsem,
    )


_scatter_hid = _make_scatter_kernel(HID, 2, nbuf=2, lookahead=1)
_scatter_f2 = _make_scatter_kernel(F2, 4)


RBLK = 2000
GRID = N // RBLK


def _dinv_from(degp_ref):
    deg = degp_ref[0, :, 0:1] + degp_ref[1, :, 0:1] + 1.0
    return lax.rsqrt(deg)


def _tc1_body(x_ref, w1_ref, degp_ref, g1_ref):
    dinv = _dinv_from(degp_ref)
    h = jnp.dot(x_ref[...], w1_ref[...], preferred_element_type=jnp.float32)
    g1_ref[...] = h * dinv


_tc1 = pl.pallas_call(
    _tc1_body,
    grid=(GRID,),
    in_specs=[
        pl.BlockSpec((RBLK, D_IN), lambda i: (i, 0)),
        pl.BlockSpec((D_IN, HID), lambda i: (0, 0)),
        pl.BlockSpec((NC, RBLK, 16), lambda i: (0, i, 0)),
    ],
    out_specs=pl.BlockSpec((RBLK, HID), lambda i: (i, 0)),
    out_shape=jax.ShapeDtypeStruct((N, HID), jnp.float32),
)


def _tc2_body(p_ref, g1_ref, degp_ref, b1_ref, w2_ref, g2_ref):
    dinv = _dinv_from(degp_ref)
    agg = p_ref[0] + p_ref[1] + g1_ref[...]
    h1 = jnp.maximum(dinv * agg + b1_ref[...], 0.0)
    h2 = jnp.dot(h1, w2_ref[...], preferred_element_type=jnp.float32)
    g2_ref[...] = h2 * dinv


_tc2 = pl.pallas_call(
    _tc2_body,
    grid=(GRID,),
    in_specs=[
        pl.BlockSpec((NC, RBLK, HID), lambda i: (0, i, 0)),
        pl.BlockSpec((RBLK, HID), lambda i: (i, 0)),
        pl.BlockSpec((NC, RBLK, 16), lambda i: (0, i, 0)),
        pl.BlockSpec((1, HID), lambda i: (0, 0)),
        pl.BlockSpec((HID, F2), lambda i: (0, 0)),
    ],
    out_specs=pl.BlockSpec((RBLK, F2), lambda i: (i, 0)),
    out_shape=jax.ShapeDtypeStruct((N, F2), jnp.float32),
)


def _tc3_body(p_ref, g2_ref, degp_ref, b2_ref, out_ref):
    dinv = _dinv_from(degp_ref)
    z = dinv * (p_ref[0] + p_ref[1] + g2_ref[...]) + b2_ref[...]
    z = z[:, :NCLS]
    m = jnp.max(z, axis=1, keepdims=True)
    zs = z - m
    out_ref[...] = zs - jnp.log(jnp.sum(jnp.exp(zs), axis=1, keepdims=True))


_tc3 = pl.pallas_call(
    _tc3_body,
    grid=(GRID,),
    in_specs=[
        pl.BlockSpec((NC, RBLK, F2), lambda i: (0, i, 0)),
        pl.BlockSpec((RBLK, F2), lambda i: (i, 0)),
        pl.BlockSpec((NC, RBLK, 16), lambda i: (0, i, 0)),
        pl.BlockSpec((1, F2), lambda i: (0, 0)),
    ],
    out_specs=pl.BlockSpec((RBLK, NCLS), lambda i: (i, 0)),
    out_shape=jax.ShapeDtypeStruct((N, NCLS), jnp.float32),
)


@jax.jit
def kernel(x, edge_index, W1, b1, W2, b2):
    src = edge_index[0].astype(jnp.int32)
    dst = edge_index[1].astype(jnp.int32)
    pad_dst = N + jnp.arange(EPAD - E, dtype=jnp.int32) % (NPAD - N)
    src_p = jnp.concatenate([src, jnp.zeros((EPAD - E,), jnp.int32)])
    dst_p = jnp.concatenate([dst, pad_dst])
    src_r = src_p.reshape(NW, NCHUNK, CHUNK)
    dst_r = dst_p.reshape(NW, NCHUNK, CHUNK)

    w2p = jnp.pad(W2, ((0, 0), (0, F2 - NCLS)))
    b1r = b1.reshape(1, HID)
    b2r = jnp.pad(b2, (0, F2 - NCLS)).reshape(1, F2)

    degp = _deg_kernel(dst_r)
    g1 = _tc1(x, W1, degp)
    p1 = _scatter_hid(g1, src_r, dst_r)
    g2 = _tc2(p1, g1, degp, b1r, w2p)
    p2 = _scatter_f2(g2, src_r, dst_r)
    return _tc3(p2, g2, degp, b2r)
